# baseline (speedup 1.0000x reference)
import functools

import jax
import jax.numpy as jnp
from jax import lax
from jax.experimental import pallas as pl
from jax.experimental.pallas import tpu as pltpu

_C = 21
_SMOOTH = 1e-06
_R = 64
_W = 512


def _treesum(e):
    t = e[0:32] + e[32:64]
    t = t[0:16] + t[16:32]
    return t[0:8] + t[8:16]


def _iou_body(x_ref, y_ref, o_ref, acc, *, nb, b_total):
    b = pl.program_id(0)
    j = pl.program_id(1)

    @pl.when((b == 0) & (j == 0))
    def _init():
        acc[...] = jnp.zeros_like(acc)

    xb = x_ref[0]
    yb = y_ref[0]

    m = xb[0]
    pred = jnp.zeros((_R, _W), jnp.int32)
    for c in range(1, _C):
        xc = xb[c]
        gt = xc > m
        m = jnp.where(gt, xc, m)
        pred = jnp.where(gt, c, pred)

    for c in range(_C):
        pm = pred == c
        ym = yb == c
        s1 = jnp.where(ym, 16513.0, 128.0)
        s2 = jnp.where(ym, 16384.0, 0.0)
        acc[b, c] = acc[b, c] + _treesum(jnp.where(pm, s1, s2))

    @pl.when((b == b_total - 1) & (j == nb - 1))
    def _finalize():
        av = acc[...]
        yv_e = jnp.floor(av / 16384.0)
        rem = av - yv_e * 16384.0
        p_e = jnp.floor(rem / 128.0)
        i_e = rem - p_e * 128.0
        i = jnp.sum(i_e, axis=(2, 3))
        p = jnp.sum(p_e, axis=(2, 3))
        yv = jnp.sum(yv_e, axis=(2, 3))
        u = p + yv - i
        iou = (i + _SMOOTH) / (u + _SMOOTH)
        miou = jnp.sum(iou, axis=1, keepdims=True) / _C
        loss = 1.0 - miou
        o_ref[...] = (jnp.sum(loss) / b_total).reshape(1, 1)


@jax.jit
def kernel(x, y):
    B, C, H, W = x.shape
    y = y.astype(jnp.int32)
    nb = H // _R
    out = pl.pallas_call(
        functools.partial(_iou_body, nb=nb, b_total=B),
        grid=(B, nb),
        in_specs=[
            pl.BlockSpec((1, C, _R, W), lambda b, j: (b, 0, j, 0)),
            pl.BlockSpec((1, _R, W), lambda b, j: (b, j, 0)),
        ],
        out_specs=pl.BlockSpec((1, 1), lambda b, j: (0, 0)),
        out_shape=jax.ShapeDtypeStruct((1, 1), jnp.float32),
        scratch_shapes=[
            pltpu.VMEM((B, _C, 8, W), jnp.float32),
        ],
        compiler_params=pltpu.CompilerParams(
            dimension_semantics=("arbitrary", "arbitrary"),
        ),
    )(x, y)
    return out[0, 0]


# speedup vs baseline: 1.1000x; 1.1000x over previous
import functools

import jax
import jax.numpy as jnp
from jax import lax
from jax.experimental import pallas as pl
from jax.experimental.pallas import tpu as pltpu

_C = 21
_SMOOTH = 1e-06
_R = 128
_W = 512


def _treesum(e):
    n = e.shape[0]
    while n > 8:
        h = n // 2
        e = e[0:h] + e[h:n]
        n = h
    return e


def _iou_body(x_ref, y_ref, o_ref, acc, *, nb, b_total):
    b = pl.program_id(0)
    j = pl.program_id(1)

    @pl.when((b == 0) & (j == 0))
    def _init():
        acc[...] = jnp.zeros_like(acc)

    xb = x_ref[0]
    yb = y_ref[0]

    m = xb[0]
    pred = jnp.zeros((_R, _W), jnp.int32)
    for c in range(1, _C):
        xc = xb[c]
        gt = xc > m
        m = jnp.where(gt, xc, m)
        pred = jnp.where(gt, c, pred)

    for c in range(_C):
        pm = pred == c
        ym = yb == c
        s1 = jnp.where(ym, 16513.0, 128.0)
        s2 = jnp.where(ym, 16384.0, 0.0)
        acc[b, c] = acc[b, c] + _treesum(jnp.where(pm, s1, s2))

    @pl.when((b == b_total - 1) & (j == nb - 1))
    def _finalize():
        av = acc[...]
        yv_e = jnp.floor(av / 16384.0)
        rem = av - yv_e * 16384.0
        p_e = jnp.floor(rem / 128.0)
        i_e = rem - p_e * 128.0
        i = jnp.sum(i_e, axis=(2, 3))
        p = jnp.sum(p_e, axis=(2, 3))
        yv = jnp.sum(yv_e, axis=(2, 3))
        u = p + yv - i
        iou = (i + _SMOOTH) / (u + _SMOOTH)
        miou = jnp.sum(iou, axis=1, keepdims=True) / _C
        loss = 1.0 - miou
        o_ref[...] = (jnp.sum(loss) / b_total).reshape(1, 1)


@jax.jit
def kernel(x, y):
    B, C, H, W = x.shape
    y = y.astype(jnp.int32)
    nb = H // _R
    out = pl.pallas_call(
        functools.partial(_iou_body, nb=nb, b_total=B),
        grid=(B, nb),
        in_specs=[
            pl.BlockSpec((1, C, _R, W), lambda b, j: (b, 0, j, 0)),
            pl.BlockSpec((1, _R, W), lambda b, j: (b, j, 0)),
        ],
        out_specs=pl.BlockSpec((1, 1), lambda b, j: (0, 0)),
        out_shape=jax.ShapeDtypeStruct((1, 1), jnp.float32),
        scratch_shapes=[
            pltpu.VMEM((B, _C, 8, W), jnp.float32),
        ],
        compiler_params=pltpu.CompilerParams(
            dimension_semantics=("arbitrary", "arbitrary"),
        ),
    )(x, y)
    return out[0, 0]
